# Initial kernel scaffold; baseline (speedup 1.0000x reference)
import functools

import jax
import jax.numpy as jnp
from jax import lax
from jax.experimental import pallas as pl
from jax.experimental.pallas import tpu as pltpu
from jax.experimental.pallas import tpu_sc as plsc

N = 4096
D = 128
K = 32
H1 = 514
M_OUT = 16
E = N * K

BLK_A = 256


def _topk_body(coors_ref, coorsT_ref, idx_ref, dsel_ref, dist_s):
    ci = coors_ref[...]
    acc = None
    for c in range(3):
        rel = ci[:, c:c + 1] - coorsT_ref[c:c + 1, :]
        sq = rel * rel
        acc = sq if acc is None else acc + sq
    dist_s[...] = acc

    jidx = lax.broadcasted_iota(jnp.int32, (BLK_A, N), 1)
    lanek = lax.broadcasted_iota(jnp.int32, (BLK_A, K), 1)

    def body(k, _):
        dmat = dist_s[...]
        m = jnp.min(dmat, axis=1, keepdims=True)
        cand = jnp.where(dmat == m, jidx, N)
        sel = jnp.min(cand, axis=1, keepdims=True)
        idx_ref[...] = jnp.where(lanek == k, sel, idx_ref[...])
        dsel_ref[...] = jnp.where(lanek == k, m, dsel_ref[...])
        dist_s[...] = jnp.where(jidx == sel, jnp.inf, dmat)
        return 0

    lax.fori_loop(0, K, body, 0)


def _topk(coors2, coorsT):
    return pl.pallas_call(
        _topk_body,
        grid=(N // BLK_A,),
        in_specs=[
            pl.BlockSpec((BLK_A, 3), lambda i: (i, 0)),
            pl.BlockSpec((3, N), lambda i: (0, 0)),
        ],
        out_specs=[
            pl.BlockSpec((BLK_A, K), lambda i: (i, 0)),
            pl.BlockSpec((BLK_A, K), lambda i: (i, 0)),
        ],
        out_shape=[
            jax.ShapeDtypeStruct((N, K), jnp.int32),
            jax.ShapeDtypeStruct((N, K), jnp.float32),
        ],
        scratch_shapes=[pltpu.VMEM((BLK_A, N), jnp.float32)],
    )(coors2, coorsT)


NC, NS = 2, 16
NW = NC * NS
PER_W = E // NW
CH = 128
NCH = PER_W // CH
CPAD = 16

_sc_mesh = plsc.VectorSubcoreMesh(
    core_axis_name="c", subcore_axis_name="s", num_cores=NC, num_subcores=NS)


@functools.partial(
    pl.kernel,
    out_type=[
        jax.ShapeDtypeStruct((E, D), jnp.float32),
        jax.ShapeDtypeStruct((E, CPAD), jnp.float32),
    ],
    mesh=_sc_mesh,
    scratch_types=[
        pltpu.VMEM((CH,), jnp.int32),
        pltpu.VMEM((CH, D), jnp.float32),
        pltpu.VMEM((CH, CPAD), jnp.float32),
        pltpu.SemaphoreType.DMA,
        pltpu.SemaphoreType.DMA,
    ],
)
def _sc_gather(feats_hbm, cpad_hbm, idx_hbm, outg_hbm, outc_hbm,
               idx_v, rows_v, cj_v, sem_f, sem_c):
    wid = lax.axis_index("s") * NC + lax.axis_index("c")
    base = wid * PER_W

    def chunk(c, _):
        off = base + c * CH
        pltpu.sync_copy(idx_hbm.at[pl.ds(off, CH)], idx_v)
        cp_f = pltpu.async_copy(feats_hbm.at[idx_v], rows_v, sem_f)
        cp_c = pltpu.async_copy(cpad_hbm.at[idx_v], cj_v, sem_c)
        cp_f.wait()
        cp_c.wait()
        pltpu.sync_copy(rows_v, outg_hbm.at[pl.ds(off, CH)])
        pltpu.sync_copy(cj_v, outc_hbm.at[pl.ds(off, CH)])
        return 0

    lax.fori_loop(0, NCH, chunk, 0)


BN = 32
BE = BN * K


def _fused_body(f_ref, g_ref, cj_ref, de_ref, c_ref,
                wtop_ref, wmid_ref, wlast_ref, eb1_ref, ew2_ref, eb2_ref,
                cw1_ref, cb1_ref, cw2_ref, cb2_ref,
                nw1a_ref, nw1b_ref, nb1_ref, nw2_ref, nb2_ref,
                nodeo_ref, coorso_ref):
    silu = jax.nn.silu
    f = f_ref[...]
    ai = jnp.dot(f, wtop_ref[...], preferred_element_type=jnp.float32)
    pre = jnp.dot(g_ref[...], wmid_ref[...],
                  preferred_element_type=jnp.float32)
    pre3 = pre.reshape(BN, K, H1) + ai[:, None, :]
    de3 = de_ref[...].reshape(BN, K, 1)
    pre3 = pre3 + de3 * wlast_ref[...][None] + eb1_ref[...][None]
    h = silu(pre3).reshape(BE, H1)
    m = silu(jnp.dot(h, ew2_ref[...], preferred_element_type=jnp.float32)
             + eb2_ref[...])
    chid = silu(jnp.dot(m, cw1_ref[...], preferred_element_type=jnp.float32)
                + cb1_ref[...])
    cwt = (jnp.dot(chid, cw2_ref[...], preferred_element_type=jnp.float32)
           + cb2_ref[...])
    w3 = cwt.reshape(BN, K, 1)
    cj3 = cj_ref[...].reshape(BN, K, CPAD)[:, :, :3]
    ci = c_ref[...]
    rc3 = ci[:, None, :] - cj3
    coorso_ref[...] = jnp.sum(w3 * rc3, axis=1) + ci
    m_i = jnp.sum(m.reshape(BN, K, M_OUT), axis=1)
    npre = (jnp.dot(f, nw1a_ref[...], preferred_element_type=jnp.float32)
            + jnp.dot(m_i, nw1b_ref[...], preferred_element_type=jnp.float32)
            + nb1_ref[...])
    nh = silu(npre)
    nodeo_ref[...] = (jnp.dot(nh, nw2_ref[...],
                              preferred_element_type=jnp.float32)
                      + nb2_ref[...] + f)


def _fused(feats2, g, cj, de, coors2, wtop, wmid, wlast, eb1, ew2, eb2,
           cw1, cb1, cw2, cb2, nw1a, nw1b, nb1, nw2, nb2):
    def full(shape):
        return pl.BlockSpec(shape, lambda i: tuple(0 for _ in shape))
    return pl.pallas_call(
        _fused_body,
        grid=(N // BN,),
        in_specs=[
            pl.BlockSpec((BN, D), lambda i: (i, 0)),
            pl.BlockSpec((BE, D), lambda i: (i, 0)),
            pl.BlockSpec((BE, CPAD), lambda i: (i, 0)),
            pl.BlockSpec((BE, 1), lambda i: (i, 0)),
            pl.BlockSpec((BN, 3), lambda i: (i, 0)),
            full((D, H1)), full((D, H1)), full((1, H1)), full((1, H1)),
            full((H1, M_OUT)), full((1, M_OUT)),
            full((M_OUT, 64)), full((1, 64)), full((64, 1)), full((1, 1)),
            full((D, 2 * D)), full((M_OUT, 2 * D)), full((1, 2 * D)),
            full((2 * D, D)), full((1, D)),
        ],
        out_specs=[
            pl.BlockSpec((BN, D), lambda i: (i, 0)),
            pl.BlockSpec((BN, 3), lambda i: (i, 0)),
        ],
        out_shape=[
            jax.ShapeDtypeStruct((N, D), jnp.float32),
            jax.ShapeDtypeStruct((N, 3), jnp.float32),
        ],
    )(feats2, g, cj, de, coors2, wtop, wmid, wlast, eb1, ew2, eb2,
      cw1, cb1, cw2, cb2, nw1a, nw1b, nb1, nw2, nb2)


def kernel(feats, coors, ew1, eb1, ew2, eb2, cw1, cb1, cw2, cb2,
           nw1, nb1, nw2, nb2):
    feats2 = feats[0]
    coors2 = coors[0]
    coorsT = coors2.T

    idx, dsel = _topk(coors2, coorsT)

    idx_flat = idx.reshape(E)
    cpad = jnp.concatenate(
        [coors2, jnp.zeros((N, CPAD - 3), coors2.dtype)], axis=1)
    g, cj = _sc_gather(feats2, cpad, idx_flat)

    de = dsel.reshape(E, 1)
    node_out, coors_out = _fused(
        feats2, g, cj, de, coors2,
        ew1[:D], ew1[D:2 * D], ew1[2 * D:2 * D + 1], eb1[None],
        ew2, eb2[None], cw1, cb1[None], cw2, cb2[None],
        nw1[:D], nw1[D:], nb1[None], nw2, nb2[None])
    return node_out[None], coors_out[None]


# speedup vs baseline: 7.7937x; 7.7937x over previous
import functools

import jax
import jax.numpy as jnp
from jax import lax
from jax.experimental import pallas as pl
from jax.experimental.pallas import tpu as pltpu
from jax.experimental.pallas import tpu_sc as plsc

N = 4096
D = 128
K = 32
H1 = 514
M_OUT = 16
E = N * K

BLK_A = 256


def _topk_body(coors_ref, coorsT_ref, idx_ref, dsel_ref, dist_s):
    ci = coors_ref[...]
    acc = None
    for c in range(3):
        rel = ci[:, c:c + 1] - coorsT_ref[c:c + 1, :]
        sq = rel * rel
        acc = sq if acc is None else acc + sq
    dist_s[...] = acc

    jidx = lax.broadcasted_iota(jnp.int32, (BLK_A, N), 1)
    lanek = lax.broadcasted_iota(jnp.int32, (BLK_A, K), 1)

    def body(k, _):
        dmat = dist_s[...]
        m = jnp.min(dmat, axis=1, keepdims=True)
        cand = jnp.where(dmat == m, jidx, N)
        sel = jnp.min(cand, axis=1, keepdims=True)
        idx_ref[...] = jnp.where(lanek == k, sel, idx_ref[...])
        dsel_ref[...] = jnp.where(lanek == k, m, dsel_ref[...])
        dist_s[...] = jnp.where(jidx == sel, jnp.inf, dmat)
        return 0

    lax.fori_loop(0, K, body, 0)


def _topk(coors2, coorsT):
    return pl.pallas_call(
        _topk_body,
        grid=(N // BLK_A,),
        in_specs=[
            pl.BlockSpec((BLK_A, 3), lambda i: (i, 0)),
            pl.BlockSpec((3, N), lambda i: (0, 0)),
        ],
        out_specs=[
            pl.BlockSpec((BLK_A, K), lambda i: (i, 0)),
            pl.BlockSpec((BLK_A, K), lambda i: (i, 0)),
        ],
        out_shape=[
            jax.ShapeDtypeStruct((N, K), jnp.int32),
            jax.ShapeDtypeStruct((N, K), jnp.float32),
        ],
        scratch_shapes=[pltpu.VMEM((BLK_A, N), jnp.float32)],
    )(coors2, coorsT)


NC, NS = 2, 16
NW = NC * NS
PER_W = E // NW
CH = 128
NCH = PER_W // CH
CPAD = 128

def _sc_gather_body(feats_hbm, cpad_hbm, idx_hbm, outg_hbm, outc_hbm,
                    idx_v, rows_v, cj_v, sem_f, sem_c):
    wid = lax.axis_index("s") * NC + lax.axis_index("c")
    base = wid * PER_W

    def chunk(c, _):
        off = base + c * CH
        pltpu.sync_copy(idx_hbm.at[pl.ds(off, CH)], idx_v)
        cp_f = pltpu.async_copy(feats_hbm.at[idx_v], rows_v, sem_f)
        cp_c = pltpu.async_copy(cpad_hbm.at[idx_v], cj_v, sem_c)
        cp_f.wait()
        cp_c.wait()
        pltpu.sync_copy(rows_v, outg_hbm.at[pl.ds(off, CH)])
        pltpu.sync_copy(cj_v, outc_hbm.at[pl.ds(off, CH)])
        return 0

    lax.fori_loop(0, NCH, chunk, 0)


@functools.cache
def _sc_gather_fn():
    mesh = plsc.VectorSubcoreMesh(
        core_axis_name="c", subcore_axis_name="s",
        num_cores=NC, num_subcores=NS)
    return pl.kernel(
        _sc_gather_body,
        out_type=[
            jax.ShapeDtypeStruct((E, D), jnp.float32),
            jax.ShapeDtypeStruct((E, CPAD), jnp.float32),
        ],
        mesh=mesh,
        scratch_types=[
            pltpu.VMEM((CH,), jnp.int32),
            pltpu.VMEM((CH, D), jnp.float32),
            pltpu.VMEM((CH, CPAD), jnp.float32),
            pltpu.SemaphoreType.DMA,
            pltpu.SemaphoreType.DMA,
        ],
    )


def _sc_gather(feats2, cpad, idx_flat):
    return _sc_gather_fn()(feats2, cpad, idx_flat)


BN = 32
BE = BN * K


def _fused_body(f_ref, g_ref, cj_ref, de_ref, c_ref,
                wtop_ref, wmid_ref, wlast_ref, eb1_ref, ew2_ref, eb2_ref,
                cw1_ref, cb1_ref, cw2_ref, cb2_ref,
                nw1a_ref, nw1b_ref, nb1_ref, nw2_ref, nb2_ref,
                nodeo_ref, coorso_ref):
    silu = jax.nn.silu
    f = f_ref[...]
    ai = jnp.dot(f, wtop_ref[...], preferred_element_type=jnp.float32)
    pre = jnp.dot(g_ref[...], wmid_ref[...],
                  preferred_element_type=jnp.float32)
    pre3 = pre.reshape(BN, K, H1) + ai[:, None, :]
    de3 = de_ref[...].reshape(BN, K, 1)
    pre3 = pre3 + de3 * wlast_ref[...][None] + eb1_ref[...][None]
    h = silu(pre3).reshape(BE, H1)
    m = silu(jnp.dot(h, ew2_ref[...], preferred_element_type=jnp.float32)
             + eb2_ref[...])
    chid = silu(jnp.dot(m, cw1_ref[...], preferred_element_type=jnp.float32)
                + cb1_ref[...])
    cwt = (jnp.dot(chid, cw2_ref[...], preferred_element_type=jnp.float32)
           + cb2_ref[...])
    w3 = cwt.reshape(BN, K, 1)
    cj3 = cj_ref[...].reshape(BN, K, CPAD)[:, :, :3]
    ci = c_ref[...]
    rc3 = ci[:, None, :] - cj3
    coorso_ref[...] = jnp.sum(w3 * rc3, axis=1) + ci
    m_i = jnp.sum(m.reshape(BN, K, M_OUT), axis=1)
    npre = (jnp.dot(f, nw1a_ref[...], preferred_element_type=jnp.float32)
            + jnp.dot(m_i, nw1b_ref[...], preferred_element_type=jnp.float32)
            + nb1_ref[...])
    nh = silu(npre)
    nodeo_ref[...] = (jnp.dot(nh, nw2_ref[...],
                              preferred_element_type=jnp.float32)
                      + nb2_ref[...] + f)


def _fused(feats2, g, cj, de, coors2, wtop, wmid, wlast, eb1, ew2, eb2,
           cw1, cb1, cw2, cb2, nw1a, nw1b, nb1, nw2, nb2):
    def full(shape):
        return pl.BlockSpec(shape, lambda i: tuple(0 for _ in shape))
    return pl.pallas_call(
        _fused_body,
        grid=(N // BN,),
        in_specs=[
            pl.BlockSpec((BN, D), lambda i: (i, 0)),
            pl.BlockSpec((BE, D), lambda i: (i, 0)),
            pl.BlockSpec((BE, CPAD), lambda i: (i, 0)),
            pl.BlockSpec((BE, 1), lambda i: (i, 0)),
            pl.BlockSpec((BN, 3), lambda i: (i, 0)),
            full((D, H1)), full((D, H1)), full((1, H1)), full((1, H1)),
            full((H1, M_OUT)), full((1, M_OUT)),
            full((M_OUT, 64)), full((1, 64)), full((64, 1)), full((1, 1)),
            full((D, 2 * D)), full((M_OUT, 2 * D)), full((1, 2 * D)),
            full((2 * D, D)), full((1, D)),
        ],
        out_specs=[
            pl.BlockSpec((BN, D), lambda i: (i, 0)),
            pl.BlockSpec((BN, 3), lambda i: (i, 0)),
        ],
        out_shape=[
            jax.ShapeDtypeStruct((N, D), jnp.float32),
            jax.ShapeDtypeStruct((N, 3), jnp.float32),
        ],
    )(feats2, g, cj, de, coors2, wtop, wmid, wlast, eb1, ew2, eb2,
      cw1, cb1, cw2, cb2, nw1a, nw1b, nb1, nw2, nb2)


def kernel(feats, coors, ew1, eb1, ew2, eb2, cw1, cb1, cw2, cb2,
           nw1, nb1, nw2, nb2):
    feats2 = feats[0]
    coors2 = coors[0]
    coorsT = coors2.T

    idx, dsel = _topk(coors2, coorsT)

    idx_flat = idx.reshape(E)
    cpad = jnp.concatenate(
        [coors2, jnp.zeros((N, CPAD - 3), coors2.dtype)], axis=1)
    g, cj = _sc_gather(feats2, cpad, idx_flat)

    de = dsel.reshape(E, 1)
    node_out, coors_out = _fused(
        feats2, g, cj, de, coors2,
        ew1[:D], ew1[D:2 * D], ew1[2 * D:2 * D + 1], eb1[None],
        ew2, eb2[None], cw1, cb1[None], cw2, cb2[None],
        nw1[:D], nw1[D:], nb1[None], nw2, nb2[None])
    return node_out[None], coors_out[None]
